# baseline (speedup 1.0000x reference)
import functools

import jax
import jax.numpy as jnp
from jax import lax
from jax.experimental import pallas as pl
from jax.experimental.pallas import tpu as pltpu
from jax.experimental.pallas import tpu_sc as plsc

_NC = 2
_NS = 16
_NW = _NC * _NS
_L = 16


def _make_gather(BT, V, D, T):
    b_per_w = BT // _NW
    mesh = plsc.VectorSubcoreMesh(core_axis_name="c", subcore_axis_name="s")

    @functools.partial(
        pl.kernel,
        out_type=jax.ShapeDtypeStruct((BT, D), jnp.float32),
        mesh=mesh,
        scratch_types=[
            pltpu.VMEM((b_per_w,), jnp.int32),
            pltpu.VMEM((b_per_w, D), jnp.float32),
            pltpu.VMEM((T, D), jnp.float32),
            pltpu.SemaphoreType.DMA,
        ],
        compiler_params=pltpu.CompilerParams(use_tc_tiling_on_sc=False),
    )
    def gather_k(idx_hbm, table_hbm, pos_hbm, x_hbm, idx_v, rows_v, pos_v, sem):
        wid = lax.axis_index("s") * _NC + lax.axis_index("c")
        base = wid * b_per_w
        pltpu.sync_copy(idx_hbm.at[pl.ds(base, b_per_w)], idx_v)
        pltpu.sync_copy(pos_hbm, pos_v)
        pltpu.async_copy(table_hbm.at[idx_v], rows_v, sem).wait()
        for t in range(T):
            for c in range(D // _L):
                pv = pos_v[t, pl.ds(c * _L, _L)]

                def body(g, _, t=t, c=c, pv=pv):
                    r = g * T + t
                    rows_v[r, pl.ds(c * _L, _L)] += pv
                    return 0

                lax.fori_loop(0, b_per_w // T, body, 0)
        pltpu.sync_copy(rows_v, x_hbm.at[pl.ds(base, b_per_w)])

    return gather_k


def _mm_body(bm, bn, T, x_ref, w_ref, b_ref, o_ref):
    acc = (
        jnp.dot(x_ref[...], w_ref[...], preferred_element_type=jnp.float32)
        + b_ref[...]
    )
    o_ref[...] = acc.reshape(bm // T, T, bn)


def _make_matmul(M, K, N, T, bm, bn):
    nb = M // bm
    nv = pl.cdiv(N, bn)
    return pl.pallas_call(
        functools.partial(_mm_body, bm, bn, T),
        grid=(nv, nb),
        in_specs=[
            pl.BlockSpec((bm, K), lambda j, i: (i, 0)),
            pl.BlockSpec((K, bn), lambda j, i: (0, j)),
            pl.BlockSpec((1, bn), lambda j, i: (0, j)),
        ],
        out_specs=pl.BlockSpec((bm // T, T, bn), lambda j, i: (i, 0, j)),
        out_shape=jax.ShapeDtypeStruct((M // T, T, N), jnp.float32),
        compiler_params=pltpu.CompilerParams(
            dimension_semantics=("arbitrary", "arbitrary"),
        ),
    )


def kernel(idx, tok_table, pos_table, W, b):
    B, T = idx.shape
    V, D = tok_table.shape
    N = W.shape[1]
    BT = B * T
    x = _make_gather(BT, V, D, T)(
        idx.reshape(-1).astype(jnp.int32), tok_table, pos_table
    )
    return _make_matmul(BT, D, N, T, bm=512, bn=2048)(x, W, b.reshape(1, N))


# speedup vs baseline: 1.0636x; 1.0636x over previous
import functools

import jax
import jax.numpy as jnp
from jax import lax
from jax.experimental import pallas as pl
from jax.experimental.pallas import tpu as pltpu
from jax.experimental.pallas import tpu_sc as plsc

_NC = 2
_NS = 16
_NW = _NC * _NS
_L = 16


def _make_gather(BT, V, D, T):
    b_per_w = BT // _NW
    mesh = plsc.VectorSubcoreMesh(core_axis_name="c", subcore_axis_name="s")

    @functools.partial(
        pl.kernel,
        out_type=jax.ShapeDtypeStruct((BT, D), jnp.float32),
        mesh=mesh,
        scratch_types=[
            pltpu.VMEM((b_per_w,), jnp.int32),
            pltpu.VMEM((b_per_w, D), jnp.float32),
            pltpu.VMEM((T, D), jnp.float32),
            pltpu.SemaphoreType.DMA,
        ],
        compiler_params=pltpu.CompilerParams(use_tc_tiling_on_sc=False),
    )
    def gather_k(idx_hbm, table_hbm, pos_hbm, x_hbm, idx_v, rows_v, pos_v, sem):
        wid = lax.axis_index("s") * _NC + lax.axis_index("c")
        base = wid * b_per_w
        pltpu.sync_copy(idx_hbm.at[pl.ds(base, b_per_w)], idx_v)
        pltpu.sync_copy(pos_hbm, pos_v)
        pltpu.async_copy(table_hbm.at[idx_v], rows_v, sem).wait()
        for t in range(T):
            for c in range(D // _L):
                pv = pos_v[t, pl.ds(c * _L, _L)]

                def body(g, _, t=t, c=c, pv=pv):
                    r = g * T + t
                    rows_v[r, pl.ds(c * _L, _L)] += pv
                    return 0

                lax.fori_loop(0, b_per_w // T, body, 0)
        pltpu.sync_copy(rows_v, x_hbm.at[pl.ds(base, b_per_w)])

    return gather_k


def _mm_body(bm, bn, T, x_ref, w_ref, b_ref, o_ref):
    acc = (
        jnp.dot(x_ref[...], w_ref[...], preferred_element_type=jnp.float32)
        + b_ref[...]
    )
    o_ref[...] = acc.reshape(bm // T, T, bn)


def _make_matmul(M, K, N, T, bm, bn):
    nb = M // bm
    nv = pl.cdiv(N, bn)
    return pl.pallas_call(
        functools.partial(_mm_body, bm, bn, T),
        grid=(nv, nb),
        in_specs=[
            pl.BlockSpec((bm, K), lambda j, i: (i, 0)),
            pl.BlockSpec((K, bn), lambda j, i: (0, j)),
            pl.BlockSpec((1, bn), lambda j, i: (0, j)),
        ],
        out_specs=pl.BlockSpec((bm // T, T, bn), lambda j, i: (i, 0, j)),
        out_shape=jax.ShapeDtypeStruct((M // T, T, N), jnp.float32),
        compiler_params=pltpu.CompilerParams(
            dimension_semantics=("arbitrary", "arbitrary"),
        ),
    )


def kernel(idx, tok_table, pos_table, W, b):
    B, T = idx.shape
    V, D = tok_table.shape
    N = W.shape[1]
    BT = B * T
    x = _make_gather(BT, V, D, T)(
        idx.reshape(-1).astype(jnp.int32), tok_table, pos_table
    )
    return _make_matmul(BT, D, N, T, bm=512, bn=8192)(x, W, b.reshape(1, N))


# speedup vs baseline: 3.7716x; 3.5463x over previous
import functools

import jax
import jax.numpy as jnp
from jax import lax
from jax.experimental import pallas as pl
from jax.experimental.pallas import tpu as pltpu
from jax.experimental.pallas import tpu_sc as plsc

_NC = 2
_NS = 16
_NW = _NC * _NS
_L = 16


def _make_gather(BT, V, D, T):
    b_per_w = BT // _NW
    mesh = plsc.VectorSubcoreMesh(core_axis_name="c", subcore_axis_name="s")

    @functools.partial(
        pl.kernel,
        out_type=jax.ShapeDtypeStruct((BT, D), jnp.float32),
        mesh=mesh,
        scratch_types=[
            pltpu.VMEM((b_per_w,), jnp.int32),
            pltpu.VMEM((b_per_w, D), jnp.float32),
            pltpu.VMEM((T, D), jnp.float32),
            pltpu.SemaphoreType.DMA,
        ],
        compiler_params=pltpu.CompilerParams(use_tc_tiling_on_sc=False),
    )
    def gather_k(idx_hbm, table_hbm, pos_hbm, x_hbm, idx_v, rows_v, pos_v, sem):
        wid = lax.axis_index("s") * _NC + lax.axis_index("c")
        base = wid * b_per_w
        pltpu.sync_copy(idx_hbm.at[pl.ds(base, b_per_w)], idx_v)
        pltpu.sync_copy(pos_hbm, pos_v)
        pltpu.async_copy(table_hbm.at[idx_v], rows_v, sem).wait()
        for t in range(T):
            for c in range(D // _L):
                pv = pos_v[t, pl.ds(c * _L, _L)]

                def body(g, _, t=t, c=c, pv=pv):
                    r = g * T + t
                    rows_v[r, pl.ds(c * _L, _L)] += pv
                    return 0

                lax.fori_loop(0, b_per_w // T, body, 0)
        pltpu.sync_copy(rows_v, x_hbm.at[pl.ds(base, b_per_w)])

    return gather_k


def _mmT_body(T, x_ref, w_ref, b_ref, o_ref):
    bias = b_ref[...]
    wt = w_ref[...]
    for t in range(T):
        acc = jnp.dot(wt, x_ref[t], preferred_element_type=jnp.float32)
        o_ref[t] = acc + bias


def _make_matmul(B, K, N, T, bn):
    nv = pl.cdiv(N, bn)
    return pl.pallas_call(
        functools.partial(_mmT_body, T),
        grid=(nv,),
        in_specs=[
            pl.BlockSpec((T, K, B), lambda j: (0, 0, 0)),
            pl.BlockSpec((bn, K), lambda j: (j, 0)),
            pl.BlockSpec((bn, 1), lambda j: (j, 0)),
        ],
        out_specs=pl.BlockSpec((T, bn, B), lambda j: (0, j, 0)),
        out_shape=jax.ShapeDtypeStruct((T, N, B), jnp.float32),
        compiler_params=pltpu.CompilerParams(
            dimension_semantics=("arbitrary",),
        ),
    )


def kernel(idx, tok_table, pos_table, W, b):
    B, T = idx.shape
    V, D = tok_table.shape
    N = W.shape[1]
    BT = B * T
    x = _make_gather(BT, V, D, T)(
        idx.reshape(-1).astype(jnp.int32), tok_table, pos_table
    )
    xt = x.reshape(B, T, D).transpose(1, 2, 0)
    out_t = _make_matmul(B, D, N, T, bn=400)(xt, W.T, b.reshape(N, 1))
    return out_t.transpose(2, 0, 1)


# speedup vs baseline: 3.9380x; 1.0441x over previous
import functools

import jax
import jax.numpy as jnp
from jax import lax
from jax.experimental import pallas as pl
from jax.experimental.pallas import tpu as pltpu
from jax.experimental.pallas import tpu_sc as plsc

_NC = 2
_NS = 16
_NW = _NC * _NS
_L = 16


def _make_gather(BT, V, D, T):
    b_per_w = BT // _NW
    mesh = plsc.VectorSubcoreMesh(core_axis_name="c", subcore_axis_name="s")

    @functools.partial(
        pl.kernel,
        out_type=jax.ShapeDtypeStruct((BT, D), jnp.float32),
        mesh=mesh,
        scratch_types=[
            pltpu.VMEM((b_per_w,), jnp.int32),
            pltpu.VMEM((b_per_w, D), jnp.float32),
            pltpu.VMEM((T, D), jnp.float32),
            pltpu.SemaphoreType.DMA,
        ],
        compiler_params=pltpu.CompilerParams(use_tc_tiling_on_sc=False),
    )
    def gather_k(idx_hbm, table_hbm, pos_hbm, x_hbm, idx_v, rows_v, pos_v, sem):
        wid = lax.axis_index("s") * _NC + lax.axis_index("c")
        base = wid * b_per_w
        pltpu.sync_copy(idx_hbm.at[pl.ds(base, b_per_w)], idx_v)
        pltpu.sync_copy(pos_hbm, pos_v)
        pltpu.async_copy(table_hbm.at[idx_v], rows_v, sem).wait()
        for t in range(T):
            for c in range(D // _L):
                pv = pos_v[t, pl.ds(c * _L, _L)]

                def body(g, _, t=t, c=c, pv=pv):
                    r = g * T + t
                    rows_v[r, pl.ds(c * _L, _L)] += pv
                    return 0

                lax.fori_loop(0, b_per_w // T, body, 0)
        pltpu.sync_copy(rows_v, x_hbm.at[pl.ds(base, b_per_w)])

    return gather_k


def _mmT_body(T, x_ref, w_ref, o_ref):
    w = w_ref[...]
    for t in range(T):
        o_ref[t] = lax.dot_general(
            w,
            x_ref[t],
            dimension_numbers=(((0,), (0,)), ((), ())),
            preferred_element_type=jnp.float32,
        )


def _make_matmul(B, K, N, T, bn):
    nv = pl.cdiv(N, bn)
    return pl.pallas_call(
        functools.partial(_mmT_body, T),
        grid=(nv,),
        in_specs=[
            pl.BlockSpec((T, K, B), lambda j: (0, 0, 0)),
            pl.BlockSpec((K, bn), lambda j: (0, j)),
        ],
        out_specs=pl.BlockSpec((T, bn, B), lambda j: (0, j, 0)),
        out_shape=jax.ShapeDtypeStruct((T, N, B), jnp.float32),
        compiler_params=pltpu.CompilerParams(
            dimension_semantics=("arbitrary",),
        ),
    )


def kernel(idx, tok_table, pos_table, W, b):
    B, T = idx.shape
    V, D = tok_table.shape
    N = W.shape[1]
    BT = B * T
    x = _make_gather(BT, V, D, T)(
        idx.reshape(-1).astype(jnp.int32), tok_table, pos_table
    )
    xt = x.reshape(B, T, D).transpose(1, 2, 0)
    xt_aug = jnp.concatenate([xt, jnp.ones((T, 1, B), jnp.float32)], axis=1)
    w_aug = jnp.concatenate([W, b[None, :]], axis=0)
    out_t = _make_matmul(B, D + 1, N, T, bn=512)(xt_aug, w_aug)
    return out_t.transpose(2, 0, 1)
